# baseline (speedup 1.0000x reference)
import functools

import jax
import jax.numpy as jnp
from jax import lax
from jax.experimental import pallas as pl
from jax.experimental.pallas import tpu as pltpu
from jax.experimental.pallas import tpu_sc as plsc

CHAR_DIM = 32
WORD_DIM = 32
OUT_DIM = CHAR_DIM + WORD_DIM
BSLAB = 128
L_IDX = 8
L_OUT = 2
LANES = 16
KV = BSLAB // LANES


@functools.lru_cache(maxsize=None)
def _make_sc_kernel(n_batch: int, seq_len: int):
    info = plsc.get_sparse_core_info()
    nw = info.num_cores * info.num_subcores
    assert n_batch == nw * BSLAB
    assert seq_len % L_IDX == 0 and L_IDX % L_OUT == 0
    nc = info.num_cores

    mesh = plsc.VectorSubcoreMesh(core_axis_name="c", subcore_axis_name="s")

    @functools.partial(
        pl.kernel,
        mesh=mesh,
        out_type=jax.ShapeDtypeStruct((seq_len, OUT_DIM, n_batch),
                                      jnp.float32),
        compiler_params=pltpu.CompilerParams(use_tc_tiling_on_sc=False,
                                             needs_layout_passes=False),
        scratch_types=[
            pltpu.VMEM((2, L_IDX, BSLAB), jnp.int32),
            pltpu.VMEM((L_IDX * 2 * BSLAB, CHAR_DIM), jnp.float32),
            pltpu.VMEM((2, OUT_DIM, BSLAB + 1), jnp.float32),
            pltpu.SemaphoreType.DMA((L_IDX,)),
            pltpu.SemaphoreType.DMA((2,)),
        ],
    )
    def k(voc_hbm, char_hbm, word_hbm, out_hbm, iv_v, gb_v, ob_v,
          sem_g, sem_w):
        wid = lax.axis_index("s") * nc + lax.axis_index("c")
        b0 = wid * BSLAB
        lanev = lax.iota(jnp.int32, LANES)
        rowvs = [lanev + (LANES * kk) for kk in range(KV)]
        colvs = [jnp.full((LANES,), d, dtype=jnp.int32)
                 for d in range(CHAR_DIM)]

        def gather_pair(l_local):
            return [
                pltpu.make_async_copy(
                    char_hbm.at[iv_v.at[0, l_local]],
                    gb_v.at[pl.ds(l_local * (2 * BSLAB), BSLAB)],
                    sem_g.at[lax.rem(l_local, L_IDX)]),
                pltpu.make_async_copy(
                    word_hbm.at[iv_v.at[1, l_local]],
                    gb_v.at[pl.ds(l_local * (2 * BSLAB) + BSLAB, BSLAB)],
                    sem_g.at[lax.rem(l_local, L_IDX)]),
            ]

        def write_buf(l, buf):
            return pltpu.make_async_copy(
                ob_v.at[buf, :, pl.ds(0, BSLAB)],
                out_hbm.at[l, :, pl.ds(b0, BSLAB)],
                sem_w.at[buf])

        def body(l, carry):
            l_local = lax.rem(l, L_IDX)
            buf = lax.rem(l, 2)

            @pl.when(l_local == 0)
            def _stage_superchunk():
                pltpu.sync_copy(
                    voc_hbm.at[:, pl.ds(l, L_IDX), pl.ds(b0, BSLAB)],
                    iv_v)
                for ll in range(L_IDX):
                    for c in gather_pair(ll):
                        c.start()

            @pl.when(l >= 2)
            def _drain_prev_write():
                write_buf(l - 2, buf).wait()

            for c in gather_pair(l_local):
                c.wait()

            base = l_local * (2 * BSLAB)
            bufv = jnp.full((LANES,), buf, dtype=jnp.int32)
            dvecs = [[lanev + (p * CHAR_DIM + h * LANES) for h in range(2)]
                     for p in range(2)]
            for t in range(BSLAB):
                tv = jnp.full((LANES,), t, dtype=jnp.int32)
                for p in range(2):
                    row = base + p * BSLAB + t
                    for h in range(2):
                        v = gb_v[row, pl.ds(h * LANES, LANES)]
                        plsc.store_scatter(
                            ob_v, [bufv, dvecs[p][h], tv], v)

            write_buf(l, buf).start()
            return carry

        lax.fori_loop(0, seq_len, body, 0)
        write_buf(seq_len - 2, 0).wait()
        write_buf(seq_len - 1, 1).wait()

    return k


def kernel(voc, char_table, word_table):
    b, _, l = voc.shape
    if voc.dtype != jnp.int32:
        voc = voc.astype(jnp.int32)
    voc_t = jnp.transpose(voc, (1, 2, 0))
    out_t = _make_sc_kernel(b, l)(voc_t, char_table, word_table)
    return jnp.transpose(out_t, (2, 0, 1))


# speedup vs baseline: 1.0785x; 1.0785x over previous
import functools

import jax
import jax.numpy as jnp
from jax import lax
from jax.experimental import pallas as pl
from jax.experimental.pallas import tpu as pltpu
from jax.experimental.pallas import tpu_sc as plsc

CHAR_DIM = 32
WORD_DIM = 32
OUT_DIM = CHAR_DIM + WORD_DIM
IPR = 128
NB = 2
NSLOTS = 3


@functools.lru_cache(maxsize=None)
def _make_sc_kernel(n_batch: int, seq_len: int):
    info = plsc.get_sparse_core_info()
    nw = info.num_cores * info.num_subcores
    assert n_batch % (nw * NB) == 0
    batches_per_w = n_batch // nw
    n_iter = batches_per_w // NB
    assert n_iter >= NSLOTS
    nc = info.num_cores
    chunk_tok = NB * seq_len
    splits = []
    off = 0
    while off < seq_len:
        splits.append((off, min(IPR, seq_len - off)))
        off += min(IPR, seq_len - off)

    mesh = plsc.VectorSubcoreMesh(core_axis_name="c", subcore_axis_name="s")

    @functools.partial(
        pl.kernel,
        mesh=mesh,
        out_type=jax.ShapeDtypeStruct((n_batch, seq_len, OUT_DIM), jnp.float32),
        compiler_params=pltpu.CompilerParams(use_tc_tiling_on_sc=False),
        scratch_types=[
            pltpu.VMEM((NSLOTS, NB, 2, seq_len), jnp.int32),
            pltpu.VMEM((NSLOTS, NB, seq_len, CHAR_DIM), jnp.float32),
            pltpu.VMEM((NSLOTS, NB, seq_len, WORD_DIM), jnp.float32),
            pltpu.SemaphoreType.DMA((NSLOTS,)),
            pltpu.SemaphoreType.DMA((NSLOTS,)),
        ],
    )
    def k(voc_hbm, char_hbm, word_hbm, out_hbm, iv_v, cb_v, wb_v,
          sem_g, sem_w):
        wid = lax.axis_index("s") * nc + lax.axis_index("c")
        batch_base = wid * batches_per_w

        def gather_copies(slot):
            copies = []
            for b in range(NB):
                for (o, n) in splits:
                    copies.append(pltpu.make_async_copy(
                        char_hbm.at[iv_v.at[slot, b, 0, pl.ds(o, n)]],
                        cb_v.at[slot, b, pl.ds(o, n)],
                        sem_g.at[slot]))
                    copies.append(pltpu.make_async_copy(
                        word_hbm.at[iv_v.at[slot, b, 1, pl.ds(o, n)]],
                        wb_v.at[slot, b, pl.ds(o, n)],
                        sem_g.at[slot]))
            return copies

        def issue_gathers(chunk_i, slot):
            b0 = batch_base + chunk_i * NB
            pltpu.sync_copy(voc_hbm.at[pl.ds(b0, NB)], iv_v.at[slot])
            for c in gather_copies(slot):
                c.start()

        def write_copies(chunk_i, slot):
            b0 = batch_base + chunk_i * NB
            return [
                pltpu.make_async_copy(
                    cb_v.at[slot],
                    out_hbm.at[pl.ds(b0, NB), :, pl.ds(0, CHAR_DIM)],
                    sem_w.at[slot]),
                pltpu.make_async_copy(
                    wb_v.at[slot],
                    out_hbm.at[pl.ds(b0, NB), :, pl.ds(CHAR_DIM, WORD_DIM)],
                    sem_w.at[slot]),
            ]

        for p in range(NSLOTS - 1):
            issue_gathers(p, p)

        def body(i, carry):
            slot = lax.rem(i, NSLOTS)
            for c in gather_copies(slot):
                c.wait()
            for c in write_copies(i, slot):
                c.start()

            @pl.when(i + NSLOTS - 1 < n_iter)
            def _issue_ahead():
                nslot = lax.rem(i + NSLOTS - 1, NSLOTS)

                @pl.when(i > 0)
                def _drain_stale_write():
                    for c in write_copies(i - 1, nslot):
                        c.wait()

                issue_gathers(i + NSLOTS - 1, nslot)

            return carry

        lax.fori_loop(0, n_iter, body, 0)
        for tail in range(NSLOTS, 0, -1):
            for c in write_copies(n_iter - tail, (n_iter - tail) % NSLOTS):
                c.wait()

    return k


def kernel(voc, char_table, word_table):
    b, _, l = voc.shape
    if voc.dtype != jnp.int32:
        voc = voc.astype(jnp.int32)
    return _make_sc_kernel(b, l)(voc, char_table, word_table)
